# baseline (speedup 1.0000x reference)
import functools
import jax
import jax.numpy as jnp
from jax import lax
from jax.experimental import pallas as pl
from jax.experimental.pallas import tpu as pltpu
from jax.experimental.pallas import tpu_sc as plsc

H = 32
W = 32
F = 384
HW = H * W
B = 16
NC = 2
NS = 16
NW = NC * NS
CPW = (2 * F) // NW


def _sc_body(tab_hbm, out_hbm, cols_v, chunk_v, osem):
    wid = lax.axis_index("s") * NC + lax.axis_index("c")
    swid = wid % NS
    c0 = wid * CPW
    t0 = swid * CPW

    pltpu.sync_copy(tab_hbm.at[pl.ds(t0, CPW)], cols_v)

    mask = jnp.minimum(
        jnp.full((16,), NS - 1 - wid, dtype=jnp.int32), 0
    ).astype(jnp.float32) + 1.0
    mask = jnp.maximum(mask, 0.0)

    for k in range(CPW):
        a0 = cols_v[k, pl.ds(0, 16)]
        a1 = cols_v[k, pl.ds(16, 16)]
        b0 = cols_v[k, pl.ds(32, 16)]
        b1 = cols_v[k, pl.ds(48, 16)]
        for t in range(H):
            elt = b0[t] if t < 16 else b1[t - 16]
            rep = jnp.full((16,), elt, dtype=jnp.float32)
            chunk_v[k, t, pl.ds(0, 16)] = rep + mask * (a0 - rep)
            chunk_v[k, t, pl.ds(16, 16)] = rep + mask * (a1 - rep)

    out = [
        pltpu.make_async_copy(chunk_v, out_hbm.at[b, pl.ds(c0, CPW)], osem)
        for b in range(B)
    ]
    for cp in out:
        cp.start()
    for cp in out:
        cp.wait()


def kernel(x, row_embed, col_embed):
    b = x.shape[0]
    tab_t = jnp.concatenate([col_embed.T, row_embed.T], axis=1)
    mesh = plsc.VectorSubcoreMesh(core_axis_name="c", subcore_axis_name="s")
    run = functools.partial(
        pl.kernel,
        out_type=jax.ShapeDtypeStruct((b, 2 * F, H, W), jnp.float32),
        mesh=mesh,
        scratch_types=[
            pltpu.VMEM((CPW, 64), jnp.float32),
            pltpu.VMEM((CPW, H, W), jnp.float32),
            pltpu.SemaphoreType.DMA,
        ],
    )(_sc_body)
    return run(tab_t)


# speedup vs baseline: 3.5687x; 3.5687x over previous
import jax
import jax.numpy as jnp
from jax import lax
from jax.experimental import pallas as pl
from jax.experimental.pallas import tpu as pltpu

H = 32
W = 32
F = 384
HW = H * W


def _pos_body(row_ref, col_ref, out_ref, scratch):
    pid = pl.program_id(0)

    @pl.when(pid == 0)
    def _():
        col_t = col_ref[...].T
        row_t = row_ref[...].T
        lane = lax.broadcasted_iota(jnp.int32, (W, HW), 1)
        sub = lax.broadcasted_iota(jnp.int32, (W, HW), 0)
        tile_mask = (lane % W == sub).astype(jnp.float32)
        rep_mask = (lane // W == sub).astype(jnp.float32)
        scratch[:F] = jnp.dot(col_t, tile_mask,
                              precision=lax.Precision.HIGHEST,
                              preferred_element_type=jnp.float32)
        scratch[F:] = jnp.dot(row_t, rep_mask,
                              precision=lax.Precision.HIGHEST,
                              preferred_element_type=jnp.float32)

    out_ref[0] = scratch[...]


def kernel(x, row_embed, col_embed):
    b = x.shape[0]
    out = pl.pallas_call(
        _pos_body,
        grid=(b,),
        in_specs=[
            pl.BlockSpec((H, F), lambda i: (0, 0)),
            pl.BlockSpec((W, F), lambda i: (0, 0)),
        ],
        out_specs=pl.BlockSpec((1, 2 * F, HW), lambda i: (i, 0, 0)),
        out_shape=jax.ShapeDtypeStruct((b, 2 * F, HW), jnp.float32),
        scratch_shapes=[pltpu.VMEM((2 * F, HW), jnp.float32)],
    )(row_embed, col_embed)
    return out.reshape(b, 2 * F, H, W)
